# Initial kernel scaffold; baseline (speedup 1.0000x reference)
import jax
import jax.numpy as jnp
from jax.experimental import pallas as pl


def kernel(x_type, x_tok, x_small, edge_index, edge_type, batch, type_table, tok_table, msg_W, msg_b, gru_Wih, gru_Whh, gru_bih, gru_bhh, proj_W1, proj_b1, proj_W2, proj_b2):
    raise NotImplementedError("write your pallas kernel here")


# speedup vs baseline: 3.5362x; 3.5362x over previous
import functools
import jax
import jax.numpy as jnp
from jax import lax
from jax.experimental import pallas as pl
from jax.experimental.pallas import tpu as pltpu
from jax.experimental.pallas import tpu_sc as plsc

N = 50000
E = 800000
F = 96
NET = 3
STEPS = 4
NG = 64
BN = 1000
NB = N // BN
EPAD = 800768
EPT = EPAD // 16
NCH = EPT // 128
ACCR = 50048
RPT = ACCR // 16


def _sc_agg_body(g_hbm, dst_hbm, y6_hbm, m_hbm, gbuf, dbuf, ibuf, rows,
                 zbuf, acc, sem):
    core = lax.axis_index("c")
    sid = lax.axis_index("s")

    zv = jnp.zeros((16,), jnp.float32)

    def zloop(i, carry):
        zbuf[i, :] = zv
        return carry

    lax.fori_loop(0, RPT, zloop, 0)

    for j in range(3):
        c = core * 3 + j

        pltpu.sync_copy(zbuf, acc.at[pl.ds(sid * RPT, RPT)])
        plsc.subcore_barrier()

        ebase = sid * EPT

        def chunk(i, carry):
            off = ebase + i * 128
            pltpu.sync_copy(g_hbm.at[pl.ds(off, 128)], gbuf)
            pltpu.sync_copy(dst_hbm.at[pl.ds(off, 128)], dbuf)
            for k in range(8):
                ibuf[pl.ds(k * 16, 16)] = gbuf[pl.ds(k * 16, 16)] * 6 + c
            pltpu.async_copy(y6_hbm.at[ibuf], rows, sem).wait()
            pltpu.sync_copy(rows, acc.at[dbuf], add=True)
            return carry

        lax.fori_loop(0, NCH, chunk, 0)
        plsc.subcore_barrier()

        r0 = pl.multiple_of(sid * RPT, 8)

        @pl.when(sid < 15)
        def _():
            pltpu.sync_copy(acc.at[pl.ds(r0, RPT)],
                            m_hbm.at[c, pl.ds(r0, RPT)])

        @pl.when(sid == 15)
        def _():
            pltpu.sync_copy(acc.at[pl.ds(r0, N - 15 * RPT)],
                            m_hbm.at[c, pl.ds(r0, N - 15 * RPT)])

        plsc.subcore_barrier()


_sc_mesh = plsc.VectorSubcoreMesh(core_axis_name="c", subcore_axis_name="s")

_sc_agg = functools.partial(
    pl.kernel,
    mesh=_sc_mesh,
    compiler_params=pltpu.CompilerParams(use_tc_tiling_on_sc=False),
    out_type=jax.ShapeDtypeStruct((6, N, 16), jnp.float32),
    scratch_types=[
        pltpu.VMEM((128,), jnp.int32),
        pltpu.VMEM((128,), jnp.int32),
        pltpu.VMEM((128,), jnp.int32),
        pltpu.VMEM((128, 16), jnp.float32),
        pltpu.VMEM((RPT, 16), jnp.float32),
        pltpu.VMEM_SHARED((ACCR, 16), jnp.float32),
        pltpu.SemaphoreType.DMA,
    ],
)(_sc_agg_body)


def _y_body(x_ref, w_ref, b_ref, y_ref):
    x = x_ref[...]
    for et in range(NET):
        y_ref[et] = lax.dot_general(
            x, w_ref[et], (((1,), (1,)), ((), ())),
            preferred_element_type=jnp.float32) + b_ref[et]


def _gru_body(m_ref, x_ref, wih_ref, whh_ref, bih_ref, bhh_ref,
              w_ref, b_ref, xn_ref, y_ref):
    m = jnp.concatenate([m_ref[c] for c in range(6)], axis=-1)
    x = x_ref[...]
    gi = lax.dot_general(m, wih_ref[...], (((1,), (1,)), ((), ())),
                         preferred_element_type=jnp.float32) + bih_ref[...]
    gh = lax.dot_general(x, whh_ref[...], (((1,), (1,)), ((), ())),
                         preferred_element_type=jnp.float32) + bhh_ref[...]
    r = jax.nn.sigmoid(gi[:, 0:F] + gh[:, 0:F])
    z = jax.nn.sigmoid(gi[:, F:2 * F] + gh[:, F:2 * F])
    n = jnp.tanh(gi[:, 2 * F:] + r * gh[:, 2 * F:])
    xn = (1.0 - z) * n + z * x
    xn_ref[...] = xn
    for et in range(NET):
        y_ref[et] = lax.dot_general(
            xn, w_ref[et], (((1,), (1,)), ((), ())),
            preferred_element_type=jnp.float32) + b_ref[et]


def _pool_body(x_ref, b_ref, w1_ref, b1_ref, w2_ref, b2_ref, out_ref,
               sums, cnts):
    i = pl.program_id(0)

    @pl.when(i == 0)
    def _():
        sums[...] = jnp.zeros_like(sums)
        cnts[...] = jnp.zeros_like(cnts)

    bids = b_ref[0]
    gids = lax.broadcasted_iota(jnp.int32, (NG, BN), 0)
    oh = (gids == bids).astype(jnp.float32)
    sums[...] += lax.dot_general(oh, x_ref[...], (((1,), (0,)), ((), ())),
                                 preferred_element_type=jnp.float32)
    cnts[...] += jnp.broadcast_to(jnp.sum(oh, axis=1, keepdims=True),
                                  cnts.shape)

    @pl.when(i == NB - 1)
    def _():
        pooled = sums[...] / jnp.maximum(cnts[...][:, 0:F], 1.0)
        h = jax.nn.relu(
            lax.dot_general(pooled, w1_ref[...], (((1,), (1,)), ((), ())),
                            preferred_element_type=jnp.float32)
            + b1_ref[...])
        logits = lax.dot_general(h, w2_ref[...], (((1,), (1,)), ((), ())),
                                 preferred_element_type=jnp.float32)
        out_ref[...] = logits.reshape(1, NG) + b2_ref[...]


_full = lambda shp: pl.BlockSpec(shp, lambda i: tuple(0 for _ in shp))

_tc_y = pl.pallas_call(
    _y_body,
    grid=(NB,),
    in_specs=[
        pl.BlockSpec((BN, F), lambda i: (i, 0)),
        _full((NET, F, F)),
        _full((NET, 1, F)),
    ],
    out_specs=pl.BlockSpec((NET, BN, F), lambda i: (0, i, 0)),
    out_shape=jax.ShapeDtypeStruct((NET, N, F), jnp.float32),
)

_tc_gru = pl.pallas_call(
    _gru_body,
    grid=(NB,),
    in_specs=[
        pl.BlockSpec((6, BN, 16), lambda i: (0, i, 0)),
        pl.BlockSpec((BN, F), lambda i: (i, 0)),
        _full((3 * F, F)),
        _full((3 * F, F)),
        _full((1, 3 * F)),
        _full((1, 3 * F)),
        _full((NET, F, F)),
        _full((NET, 1, F)),
    ],
    out_specs=[
        pl.BlockSpec((BN, F), lambda i: (i, 0)),
        pl.BlockSpec((NET, BN, F), lambda i: (0, i, 0)),
    ],
    out_shape=[
        jax.ShapeDtypeStruct((N, F), jnp.float32),
        jax.ShapeDtypeStruct((NET, N, F), jnp.float32),
    ],
)

_tc_pool = pl.pallas_call(
    _pool_body,
    grid=(NB,),
    in_specs=[
        pl.BlockSpec((BN, F), lambda i: (i, 0)),
        pl.BlockSpec((1, 1, BN), lambda i: (i, 0, 0)),
        _full((128, F)),
        _full((1, 128)),
        _full((1, 128)),
        _full((1, 1)),
    ],
    out_specs=pl.BlockSpec((1, NG), lambda i: (0, 0)),
    out_shape=jax.ShapeDtypeStruct((1, NG), jnp.float32),
    scratch_shapes=[
        pltpu.VMEM((NG, F), jnp.float32),
        pltpu.VMEM((NG, 128), jnp.float32),
    ],
)


def kernel(x_type, x_tok, x_small, edge_index, edge_type, batch,
           type_table, tok_table, msg_W, msg_b,
           gru_Wih, gru_Whh, gru_bih, gru_bhh,
           proj_W1, proj_b1, proj_W2, proj_b2):
    te = jnp.take(type_table, x_type[:, 0], axis=0)
    to = jnp.take(tok_table, x_tok[:, 0], axis=0)
    x = jnp.concatenate([te, to, x_small], axis=-1)

    src = edge_index[0]
    dst = edge_index[1]
    g = edge_type * N + src
    gp = jnp.pad(g, (0, EPAD - E))
    dstp = jnp.pad(dst, (0, EPAD - E), constant_values=N)

    bias3 = msg_b.reshape(NET, 1, F)
    bih = gru_bih.reshape(1, 3 * F)
    bhh = gru_bhh.reshape(1, 3 * F)

    y = _tc_y(x, msg_W, bias3)
    for _ in range(STEPS):
        m6 = _sc_agg(gp, dstp, y.reshape(NET * N * 6, 16))
        x, y = _tc_gru(m6, x, gru_Wih, gru_Whh, bih, bhh, msg_W, bias3)

    logits = _tc_pool(x, batch.reshape(NB, 1, BN), proj_W1,
                      proj_b1.reshape(1, 128), proj_W2.reshape(1, 128),
                      proj_b2.reshape(1, 1))
    return logits.reshape(NG)


# speedup vs baseline: 3.5457x; 1.0027x over previous
import functools
import jax
import jax.numpy as jnp
from jax import lax
from jax.experimental import pallas as pl
from jax.experimental.pallas import tpu as pltpu
from jax.experimental.pallas import tpu_sc as plsc

N = 50000
E = 800000
F = 96
NET = 3
STEPS = 4
NG = 64
BN = 1000
NB = N // BN
EPAD = 800768
EPT = EPAD // 16
NCH = EPT // 128
ACCR = 50048
RPT = ACCR // 16


def _sc_agg_body(g_hbm, dst_hbm, y6_hbm, m_hbm, gbuf, dbuf, ibuf, rows,
                 zbuf, acc, sem):
    core = lax.axis_index("c")
    sid = lax.axis_index("s")

    zv = jnp.zeros((16,), jnp.float32)

    def zloop(i, carry):
        zbuf[i, :] = zv
        return carry

    lax.fori_loop(0, RPT, zloop, 0)

    for j in range(3):
        c = core * 3 + j

        pltpu.sync_copy(zbuf, acc.at[pl.ds(sid * RPT, RPT)])
        plsc.subcore_barrier()

        ebase = sid * EPT

        def chunk(i, carry):
            off = ebase + i * 128
            pltpu.sync_copy(g_hbm.at[pl.ds(off, 128)], gbuf)
            pltpu.sync_copy(dst_hbm.at[pl.ds(off, 128)], dbuf)
            for k in range(8):
                ibuf[pl.ds(k * 16, 16)] = gbuf[pl.ds(k * 16, 16)] * 6 + c
            pltpu.async_copy(y6_hbm.at[ibuf], rows, sem).wait()
            pltpu.sync_copy(rows, acc.at[dbuf], add=True)
            return carry

        lax.fori_loop(0, NCH, chunk, 0)
        plsc.subcore_barrier()

        r0 = pl.multiple_of(sid * RPT, 8)

        @pl.when(sid < 15)
        def _():
            pltpu.sync_copy(acc.at[pl.ds(r0, RPT)],
                            m_hbm.at[c, pl.ds(r0, RPT)])

        @pl.when(sid == 15)
        def _():
            pltpu.sync_copy(acc.at[pl.ds(r0, N - 15 * RPT)],
                            m_hbm.at[c, pl.ds(r0, N - 15 * RPT)])

        plsc.subcore_barrier()


_sc_mesh = plsc.VectorSubcoreMesh(core_axis_name="c", subcore_axis_name="s")

_sc_agg = functools.partial(
    pl.kernel,
    mesh=_sc_mesh,
    compiler_params=pltpu.CompilerParams(use_tc_tiling_on_sc=False),
    out_type=jax.ShapeDtypeStruct((6, N, 16), jnp.float32),
    scratch_types=[
        pltpu.VMEM((128,), jnp.int32),
        pltpu.VMEM((128,), jnp.int32),
        pltpu.VMEM((128,), jnp.int32),
        pltpu.VMEM((128, 16), jnp.float32),
        pltpu.VMEM((RPT, 16), jnp.float32),
        pltpu.VMEM_SHARED((ACCR, 16), jnp.float32),
        pltpu.SemaphoreType.DMA,
    ],
)(_sc_agg_body)


def _y_body(x_ref, w_ref, b_ref, y_ref):
    x = x_ref[...]
    for et in range(NET):
        y_ref[et] = lax.dot_general(
            x, w_ref[et], (((1,), (1,)), ((), ())),
            preferred_element_type=jnp.float32) + b_ref[et]


def _gru_body(m_ref, x_ref, wih_ref, whh_ref, bih_ref, bhh_ref,
              w_ref, b_ref, xn_ref, y_ref):
    m = jnp.concatenate([m_ref[c] for c in range(6)], axis=-1)
    x = x_ref[...]
    gi = lax.dot_general(m, wih_ref[...], (((1,), (1,)), ((), ())),
                         preferred_element_type=jnp.float32) + bih_ref[...]
    gh = lax.dot_general(x, whh_ref[...], (((1,), (1,)), ((), ())),
                         preferred_element_type=jnp.float32) + bhh_ref[...]
    r = jax.nn.sigmoid(gi[:, 0:F] + gh[:, 0:F])
    z = jax.nn.sigmoid(gi[:, F:2 * F] + gh[:, F:2 * F])
    n = jnp.tanh(gi[:, 2 * F:] + r * gh[:, 2 * F:])
    xn = (1.0 - z) * n + z * x
    xn_ref[...] = xn
    for et in range(NET):
        y_ref[et] = lax.dot_general(
            xn, w_ref[et], (((1,), (1,)), ((), ())),
            preferred_element_type=jnp.float32) + b_ref[et]


def _gru_last_body(m_ref, x_ref, wih_ref, whh_ref, bih_ref, bhh_ref,
                   xn_ref):
    m = jnp.concatenate([m_ref[c] for c in range(6)], axis=-1)
    x = x_ref[...]
    gi = lax.dot_general(m, wih_ref[...], (((1,), (1,)), ((), ())),
                         preferred_element_type=jnp.float32) + bih_ref[...]
    gh = lax.dot_general(x, whh_ref[...], (((1,), (1,)), ((), ())),
                         preferred_element_type=jnp.float32) + bhh_ref[...]
    r = jax.nn.sigmoid(gi[:, 0:F] + gh[:, 0:F])
    z = jax.nn.sigmoid(gi[:, F:2 * F] + gh[:, F:2 * F])
    n = jnp.tanh(gi[:, 2 * F:] + r * gh[:, 2 * F:])
    xn_ref[...] = (1.0 - z) * n + z * x


def _pool_body(x_ref, b_ref, w1_ref, b1_ref, w2_ref, b2_ref, out_ref,
               sums, cnts):
    i = pl.program_id(0)

    @pl.when(i == 0)
    def _():
        sums[...] = jnp.zeros_like(sums)
        cnts[...] = jnp.zeros_like(cnts)

    bids = b_ref[0]
    gids = lax.broadcasted_iota(jnp.int32, (NG, BN), 0)
    oh = (gids == bids).astype(jnp.float32)
    sums[...] += lax.dot_general(oh, x_ref[...], (((1,), (0,)), ((), ())),
                                 preferred_element_type=jnp.float32)
    cnts[...] += jnp.broadcast_to(jnp.sum(oh, axis=1, keepdims=True),
                                  cnts.shape)

    @pl.when(i == NB - 1)
    def _():
        pooled = sums[...] / jnp.maximum(cnts[...][:, 0:F], 1.0)
        h = jax.nn.relu(
            lax.dot_general(pooled, w1_ref[...], (((1,), (1,)), ((), ())),
                            preferred_element_type=jnp.float32)
            + b1_ref[...])
        logits = lax.dot_general(h, w2_ref[...], (((1,), (1,)), ((), ())),
                                 preferred_element_type=jnp.float32)
        out_ref[...] = logits.reshape(1, NG) + b2_ref[...]


_full = lambda shp: pl.BlockSpec(shp, lambda i: tuple(0 for _ in shp))

_tc_y = pl.pallas_call(
    _y_body,
    grid=(NB,),
    in_specs=[
        pl.BlockSpec((BN, F), lambda i: (i, 0)),
        _full((NET, F, F)),
        _full((NET, 1, F)),
    ],
    out_specs=pl.BlockSpec((NET, BN, F), lambda i: (0, i, 0)),
    out_shape=jax.ShapeDtypeStruct((NET, N, F), jnp.float32),
)

_tc_gru = pl.pallas_call(
    _gru_body,
    grid=(NB,),
    in_specs=[
        pl.BlockSpec((6, BN, 16), lambda i: (0, i, 0)),
        pl.BlockSpec((BN, F), lambda i: (i, 0)),
        _full((3 * F, F)),
        _full((3 * F, F)),
        _full((1, 3 * F)),
        _full((1, 3 * F)),
        _full((NET, F, F)),
        _full((NET, 1, F)),
    ],
    out_specs=[
        pl.BlockSpec((BN, F), lambda i: (i, 0)),
        pl.BlockSpec((NET, BN, F), lambda i: (0, i, 0)),
    ],
    out_shape=[
        jax.ShapeDtypeStruct((N, F), jnp.float32),
        jax.ShapeDtypeStruct((NET, N, F), jnp.float32),
    ],
)

_tc_gru_last = pl.pallas_call(
    _gru_last_body,
    grid=(NB,),
    in_specs=[
        pl.BlockSpec((6, BN, 16), lambda i: (0, i, 0)),
        pl.BlockSpec((BN, F), lambda i: (i, 0)),
        _full((3 * F, F)),
        _full((3 * F, F)),
        _full((1, 3 * F)),
        _full((1, 3 * F)),
    ],
    out_specs=pl.BlockSpec((BN, F), lambda i: (i, 0)),
    out_shape=jax.ShapeDtypeStruct((N, F), jnp.float32),
)

_tc_pool = pl.pallas_call(
    _pool_body,
    grid=(NB,),
    in_specs=[
        pl.BlockSpec((BN, F), lambda i: (i, 0)),
        pl.BlockSpec((1, 1, BN), lambda i: (i, 0, 0)),
        _full((128, F)),
        _full((1, 128)),
        _full((1, 128)),
        _full((1, 1)),
    ],
    out_specs=pl.BlockSpec((1, NG), lambda i: (0, 0)),
    out_shape=jax.ShapeDtypeStruct((1, NG), jnp.float32),
    scratch_shapes=[
        pltpu.VMEM((NG, F), jnp.float32),
        pltpu.VMEM((NG, 128), jnp.float32),
    ],
)


def kernel(x_type, x_tok, x_small, edge_index, edge_type, batch,
           type_table, tok_table, msg_W, msg_b,
           gru_Wih, gru_Whh, gru_bih, gru_bhh,
           proj_W1, proj_b1, proj_W2, proj_b2):
    te = jnp.take(type_table, x_type[:, 0], axis=0)
    to = jnp.take(tok_table, x_tok[:, 0], axis=0)
    x = jnp.concatenate([te, to, x_small], axis=-1)

    src = edge_index[0]
    dst = edge_index[1]
    g = edge_type * N + src
    gp = jnp.pad(g, (0, EPAD - E))
    dstp = jnp.pad(dst, (0, EPAD - E), constant_values=N)

    bias3 = msg_b.reshape(NET, 1, F)
    bih = gru_bih.reshape(1, 3 * F)
    bhh = gru_bhh.reshape(1, 3 * F)

    y = _tc_y(x, msg_W, bias3)
    for step in range(STEPS):
        m6 = _sc_agg(gp, dstp, y.reshape(NET * N * 6, 16))
        if step < STEPS - 1:
            x, y = _tc_gru(m6, x, gru_Wih, gru_Whh, bih, bhh, msg_W, bias3)
        else:
            x = _tc_gru_last(m6, x, gru_Wih, gru_Whh, bih, bhh)

    logits = _tc_pool(x, batch.reshape(NB, 1, BN), proj_W1,
                      proj_b1.reshape(1, 128), proj_W2.reshape(1, 128),
                      proj_b2.reshape(1, 1))
    return logits.reshape(NG)
